# Initial kernel scaffold; baseline (speedup 1.0000x reference)
import jax
import jax.numpy as jnp
from jax.experimental import pallas as pl


def kernel(x, prev_x, norm_q_g, norm_q_b, norm_k_g, norm_k_b, norm_v_g, norm_v_b, W_q, W_k, W_v, off_w, off_b, aw_w, aw_b, out_w, out_b):
    raise NotImplementedError("write your pallas kernel here")


# speedup vs baseline: 2731.5522x; 2731.5522x over previous
import functools

import jax
import jax.numpy as jnp
from jax import lax
from jax.experimental import pallas as pl
from jax.experimental.pallas import tpu as pltpu
from jax.experimental.pallas import tpu_sc as plsc

B, N, D = 2, 2048, 1024
H, P, DH = 16, 4, 64
INNER = H * DH
TWO_N = 2 * N
SCALE = DH ** -0.5

NC, NS = 2, 16
NW = NC * NS
ROWS = B * H * N * P
ROWS_PER_W = ROWS // NW
CHUNK = 128
NCHUNK = ROWS_PER_W // CHUNK

NB1 = 256
NB3 = 256
NB4 = 512


def _norm(xb):
    m = jnp.mean(xb, axis=-1, keepdims=True)
    v = jnp.mean((xb - m) ** 2, axis=-1, keepdims=True)
    return (xb - m) / jnp.sqrt(v + 1e-5)


def _dot_t(a, w):
    return lax.dot_general(a, w, (((1,), (1,)), ((), ())),
                           preferred_element_type=jnp.float32)


def _stage1_body(x_ref, px_ref, nqg_ref, nqb_ref, nkg_ref, nkb_ref,
                 nvg_ref, nvb_ref, wq_ref, wk_ref, wv_ref, ow_ref, ob_ref,
                 q_ref, off_ref, idx_ref, k_ref, v_ref):
    b = pl.program_id(0)
    nb = pl.program_id(1)
    xb = x_ref[0]
    pb = px_ref[0]

    qin = jnp.concatenate([xb, pb], axis=-1)
    qin = _norm(qin) * nqg_ref[...][None, :] + nqb_ref[...][None, :]
    q = _dot_t(qin, wq_ref[...])
    q_ref[0] = q
    off = _dot_t(q, ow_ref[...]) + ob_ref[...][None, :]
    off_ref[0] = off

    n_iota = (lax.broadcasted_iota(jnp.int32, (NB1, H * P), 0)
              + nb * NB1).astype(jnp.float32)
    sample = jnp.clip(n_iota + off, 0.0, float(TWO_N - 1))
    j = sample.astype(jnp.int32)
    m = (j >= N).astype(jnp.int32)
    pos = j - m * N
    h_lane = lax.broadcasted_iota(jnp.int32, (NB1, H * P), 1) // P
    idx_ref[0] = ((b * N + pos) * 2 + m) * H + h_lane

    xn = _norm(xb)
    pn = _norm(pb)
    nkg = nkg_ref[...][None, :]
    nkb = nkb_ref[...][None, :]
    nvg = nvg_ref[...][None, :]
    nvb = nvb_ref[...][None, :]
    k_ref[0, :, 0, :] = _dot_t(xn * nkg + nkb, wk_ref[...])
    k_ref[0, :, 1, :] = _dot_t(pn * nkg + nkb, wk_ref[...])
    v_ref[0, :, 0, :] = _dot_t(xn * nvg + nvb, wv_ref[...])
    v_ref[0, :, 1, :] = _dot_t(pn * nvg + nvb, wv_ref[...])


def _stage1(x, prev_x, nqg, nqb, nkg, nkb, nvg, nvb, W_q, W_k, W_v, off_w, off_b):
    grid = (B, N // NB1)
    full = lambda shape: pl.BlockSpec(shape, lambda b, nb: (0,) * len(shape))
    return pl.pallas_call(
        _stage1_body,
        grid=grid,
        in_specs=[
            pl.BlockSpec((1, NB1, D), lambda b, nb: (b, nb, 0)),
            pl.BlockSpec((1, NB1, D), lambda b, nb: (b, nb, 0)),
            full((2 * D,)), full((2 * D,)), full((D,)), full((D,)),
            full((D,)), full((D,)),
            full((INNER, 2 * D)), full((INNER, D)), full((INNER, D)),
            full((H * P, INNER)), full((H * P,)),
        ],
        out_specs=[
            pl.BlockSpec((1, NB1, INNER), lambda b, nb: (b, nb, 0)),
            pl.BlockSpec((1, NB1, H * P), lambda b, nb: (b, nb, 0)),
            pl.BlockSpec((1, NB1, H * P), lambda b, nb: (b, nb, 0)),
            pl.BlockSpec((1, NB1, 2, INNER), lambda b, nb: (b, nb, 0, 0)),
            pl.BlockSpec((1, NB1, 2, INNER), lambda b, nb: (b, nb, 0, 0)),
        ],
        out_shape=[
            jax.ShapeDtypeStruct((B, N, INNER), jnp.float32),
            jax.ShapeDtypeStruct((B, N, H * P), jnp.float32),
            jax.ShapeDtypeStruct((B, N, H * P), jnp.int32),
            jax.ShapeDtypeStruct((B, N, 2, INNER), jnp.float32),
            jax.ShapeDtypeStruct((B, N, 2, INNER), jnp.float32),
        ],
    )(x, prev_x, nqg, nqb, nkg, nkb, nvg, nvb, W_q, W_k, W_v, off_w, off_b)


def _gather_body(k_tab, v_tab, idx_hbm, ks_out, vs_out,
                 idx_v, kbuf, vbuf, ksem, vsem):
    wid = lax.axis_index("s") * NC + lax.axis_index("c")
    pltpu.sync_copy(idx_hbm.at[pl.ds(wid * NCHUNK, NCHUNK)], idx_v)

    def body(i, carry):
        base = wid * ROWS_PER_W + i * CHUNK
        pltpu.async_copy(k_tab.at[idx_v.at[i]], kbuf, ksem).wait()
        pltpu.sync_copy(kbuf, ks_out.at[pl.ds(base, CHUNK)])
        pltpu.async_copy(v_tab.at[idx_v.at[i]], vbuf, vsem).wait()
        pltpu.sync_copy(vbuf, vs_out.at[pl.ds(base, CHUNK)])
        return carry

    lax.fori_loop(0, NCHUNK, body, 0)


def _gather(k_tab, v_tab, idx):
    mesh = plsc.VectorSubcoreMesh(core_axis_name="c", subcore_axis_name="s",
                                  num_cores=NC, num_subcores=NS)
    f = functools.partial(
        pl.kernel,
        out_type=[
            jax.ShapeDtypeStruct((ROWS, DH), jnp.float32),
            jax.ShapeDtypeStruct((ROWS, DH), jnp.float32),
        ],
        mesh=mesh,
        scratch_types=[
            pltpu.VMEM((NCHUNK, CHUNK), jnp.int32),
            pltpu.VMEM((CHUNK, DH), jnp.float32),
            pltpu.VMEM((CHUNK, DH), jnp.float32),
            pltpu.SemaphoreType.DMA,
            pltpu.SemaphoreType.DMA,
        ],
        compiler_params=pltpu.CompilerParams(use_tc_tiling_on_sc=False),
    )(_gather_body)
    return f(k_tab.reshape(B * N * 2 * H, DH),
             v_tab.reshape(B * N * 2 * H, DH),
             idx.reshape(ROWS // CHUNK, CHUNK))


def _attn_body(q_ref, ks_ref, vs_ref, out_ref):
    q = q_ref[0]
    ks = ks_ref[0]
    vs = vs_ref[0]
    seg = (lax.broadcasted_iota(jnp.int32, (INNER, H), 0) // DH
           == lax.broadcasted_iota(jnp.int32, (INNER, H), 1)
           ).astype(jnp.float32)
    qk = q[:, None, :] * ks
    sim = lax.dot_general(qk.reshape(NB3 * P, INNER), seg,
                          (((1,), (0,)), ((), ())),
                          preferred_element_type=jnp.float32) * SCALE
    sim = sim.reshape(NB3, P, H)
    sim = sim - jnp.max(sim, axis=1, keepdims=True)
    e = jnp.exp(sim)
    attn = e / jnp.sum(e, axis=1, keepdims=True)
    a_exp = lax.dot_general(attn.reshape(NB3 * P, H), seg,
                            (((1,), (1,)), ((), ())),
                            preferred_element_type=jnp.float32)
    out_ref[0] = jnp.sum(a_exp.reshape(NB3, P, INNER) * vs, axis=1)


def _attn(q, ks, vs):
    grid = (B, N // NB3)
    return pl.pallas_call(
        _attn_body,
        grid=grid,
        in_specs=[
            pl.BlockSpec((1, NB3, INNER), lambda b, nb: (b, nb, 0)),
            pl.BlockSpec((1, NB3, P, INNER), lambda b, nb: (b, nb, 0, 0)),
            pl.BlockSpec((1, NB3, P, INNER), lambda b, nb: (b, nb, 0, 0)),
        ],
        out_specs=pl.BlockSpec((1, NB3, INNER), lambda b, nb: (b, nb, 0)),
        out_shape=jax.ShapeDtypeStruct((B, N, INNER), jnp.float32),
    )(q, ks.reshape(B, N, P, INNER), vs.reshape(B, N, P, INNER))


def _proj_body(a_ref, w_ref, b_ref, out_ref):
    out_ref[0] = _dot_t(a_ref[0], w_ref[...]) + b_ref[...][None, :]


def _proj(att, out_w, out_b):
    grid = (B, N // NB4)
    return pl.pallas_call(
        _proj_body,
        grid=grid,
        in_specs=[
            pl.BlockSpec((1, NB4, INNER), lambda b, nb: (b, nb, 0)),
            pl.BlockSpec((D, INNER), lambda b, nb: (0, 0)),
            pl.BlockSpec((D,), lambda b, nb: (0,)),
        ],
        out_specs=pl.BlockSpec((1, NB4, D), lambda b, nb: (b, nb, 0)),
        out_shape=jax.ShapeDtypeStruct((B, N, D), jnp.float32),
    )(att, out_w, out_b)


def kernel(x, prev_x, norm_q_g, norm_q_b, norm_k_g, norm_k_b, norm_v_g,
           norm_v_b, W_q, W_k, W_v, off_w, off_b, aw_w, aw_b, out_w, out_b):
    q, off, idx, k_tab, v_tab = _stage1(
        x, prev_x, norm_q_g, norm_q_b, norm_k_g, norm_k_b, norm_v_g, norm_v_b,
        W_q, W_k, W_v, off_w, off_b)
    idx_g = idx.reshape(B, N, H, P).transpose(0, 1, 3, 2)
    ks, vs = _gather(k_tab, v_tab, idx_g)
    att = _attn(q, ks, vs)
    out = _proj(att, out_w, out_b)
    offsets = off.transpose(0, 2, 1).reshape(B, H, P, N)
    return (out, offsets)


# speedup vs baseline: 3226.6695x; 1.1813x over previous
import functools

import jax
import jax.numpy as jnp
from jax import lax
from jax.experimental import pallas as pl
from jax.experimental.pallas import tpu as pltpu
from jax.experimental.pallas import tpu_sc as plsc

B, N, D = 2, 2048, 1024
H, P, DH = 16, 4, 64
INNER = H * DH
TWO_N = 2 * N
SCALE = DH ** -0.5

NC, NS = 2, 16
NW = NC * NS
ROWS = B * H * N * P
ROWS_PER_W = ROWS // NW
CHUNK = 128
NCHUNK = ROWS_PER_W // CHUNK

NB1 = 256
NB3 = 256
NB4 = 512


def _norm(xb):
    m = jnp.mean(xb, axis=-1, keepdims=True)
    v = jnp.mean((xb - m) ** 2, axis=-1, keepdims=True)
    return (xb - m) / jnp.sqrt(v + 1e-5)


def _dot_t(a, w):
    return lax.dot_general(a, w, (((1,), (1,)), ((), ())),
                           preferred_element_type=jnp.float32)


def _stage1_body(x_ref, px_ref, nqg_ref, nqb_ref, nkg_ref, nkb_ref,
                 nvg_ref, nvb_ref, wq_ref, wk_ref, wv_ref, ow_ref, ob_ref,
                 q_ref, off_ref, idx_ref, k_ref, v_ref):
    b = pl.program_id(0)
    nb = pl.program_id(1)
    xb = x_ref[0]
    pb = px_ref[0]

    qin = jnp.concatenate([xb, pb], axis=-1)
    qin = _norm(qin) * nqg_ref[...][None, :] + nqb_ref[...][None, :]
    q = _dot_t(qin, wq_ref[...])
    q_ref[0] = q
    off = _dot_t(q, ow_ref[...]) + ob_ref[...][None, :]
    off_ref[0] = off

    n_iota = (lax.broadcasted_iota(jnp.int32, (NB1, H * P), 0)
              + nb * NB1).astype(jnp.float32)
    sample = jnp.clip(n_iota + off, 0.0, float(TWO_N - 1))
    j = sample.astype(jnp.int32)
    m = (j >= N).astype(jnp.int32)
    pos = j - m * N
    h_lane = lax.broadcasted_iota(jnp.int32, (NB1, H * P), 1) % H
    idx_ref[0] = ((b * N + pos) * 2 + m) * H + h_lane

    xn = _norm(xb)
    pn = _norm(pb)
    nkg = nkg_ref[...][None, :]
    nkb = nkb_ref[...][None, :]
    nvg = nvg_ref[...][None, :]
    nvb = nvb_ref[...][None, :]
    k_ref[0, :, 0, :] = _dot_t(xn * nkg + nkb, wk_ref[...])
    k_ref[0, :, 1, :] = _dot_t(pn * nkg + nkb, wk_ref[...])
    v_ref[0, :, 0, :] = _dot_t(xn * nvg + nvb, wv_ref[...])
    v_ref[0, :, 1, :] = _dot_t(pn * nvg + nvb, wv_ref[...])


def _stage1(x, prev_x, nqg, nqb, nkg, nkb, nvg, nvb, W_q, W_k, W_v, off_w, off_b):
    grid = (B, N // NB1)
    full = lambda shape: pl.BlockSpec(shape, lambda b, nb: (0,) * len(shape))
    return pl.pallas_call(
        _stage1_body,
        grid=grid,
        in_specs=[
            pl.BlockSpec((1, NB1, D), lambda b, nb: (b, nb, 0)),
            pl.BlockSpec((1, NB1, D), lambda b, nb: (b, nb, 0)),
            full((2 * D,)), full((2 * D,)), full((D,)), full((D,)),
            full((D,)), full((D,)),
            full((INNER, 2 * D)), full((INNER, D)), full((INNER, D)),
            full((H * P, INNER)), full((H * P,)),
        ],
        out_specs=[
            pl.BlockSpec((1, NB1, INNER), lambda b, nb: (b, nb, 0)),
            pl.BlockSpec((1, NB1, H * P), lambda b, nb: (b, nb, 0)),
            pl.BlockSpec((1, NB1, H * P), lambda b, nb: (b, nb, 0)),
            pl.BlockSpec((1, NB1, 2, INNER), lambda b, nb: (b, nb, 0, 0)),
            pl.BlockSpec((1, NB1, 2, INNER), lambda b, nb: (b, nb, 0, 0)),
        ],
        out_shape=[
            jax.ShapeDtypeStruct((B, N, INNER), jnp.float32),
            jax.ShapeDtypeStruct((B, N, H * P), jnp.float32),
            jax.ShapeDtypeStruct((B, N, H * P), jnp.int32),
            jax.ShapeDtypeStruct((B, N, 2, INNER), jnp.float32),
            jax.ShapeDtypeStruct((B, N, 2, INNER), jnp.float32),
        ],
    )(x, prev_x, nqg, nqb, nkg, nkb, nvg, nvb, W_q, W_k, W_v, off_w, off_b)


G = 4
NG = ROWS_PER_W // (G * CHUNK)


NBUF = 4
NJOBS = 2 * NCHUNK
NGROUPS = NJOBS // NBUF


def _gather_body(k_tab, v_tab, idx_hbm, ks_out, vs_out,
                 idx_v, buf, gsem, wsem):
    wid = lax.axis_index("s") * NC + lax.axis_index("c")
    pltpu.sync_copy(idx_hbm.at[pl.ds(wid * NCHUNK, NCHUNK)], idx_v)
    obase = wid * NCHUNK

    def jobs(g, bank):
        out = []
        for s in range(NBUF):
            chunk = 2 * g + s // 2
            tab = k_tab if s % 2 == 0 else v_tab
            dst = ks_out if s % 2 == 0 else vs_out
            out.append((tab.at[idx_v.at[chunk]], buf.at[bank, s],
                        dst.at[obase + chunk]))
        return out

    def fire_gathers(g, bank):
        for src, b, _ in jobs(g, bank):
            pltpu.async_copy(src, b, gsem)

    def wait_gathers(g, bank):
        for src, b, _ in jobs(g, bank):
            pltpu.make_async_copy(src, b, gsem).wait()

    def fire_writes(g, bank):
        for _, b, dst in jobs(g, bank):
            pltpu.async_copy(b, dst, wsem)

    def wait_writes(g, bank):
        for _, b, dst in jobs(g, bank):
            pltpu.make_async_copy(b, dst, wsem).wait()

    fire_gathers(0, 0)

    def outer(t, carry):
        for bank in range(2):
            g = 2 * t + bank
            wait_gathers(g, bank)
            if bank == 0:
                @pl.when(t >= 1)
                def _():
                    wait_writes(g - 1, 1)
                fire_gathers(g + 1, 1)
            else:
                @pl.when(t < NGROUPS // 2 - 1)
                def _():
                    wait_writes(g - 1, 0)
                    fire_gathers(g + 1, 0)
            fire_writes(g, bank)
        return carry

    lax.fori_loop(0, NGROUPS // 2, outer, 0)
    wait_writes(NGROUPS - 2, 0)
    wait_writes(NGROUPS - 1, 1)


def _gather(k_tab, v_tab, idx):
    mesh = plsc.VectorSubcoreMesh(core_axis_name="c", subcore_axis_name="s",
                                  num_cores=NC, num_subcores=NS)
    f = functools.partial(
        pl.kernel,
        out_type=[
            jax.ShapeDtypeStruct((ROWS // CHUNK, CHUNK, DH), jnp.float32),
            jax.ShapeDtypeStruct((ROWS // CHUNK, CHUNK, DH), jnp.float32),
        ],
        mesh=mesh,
        scratch_types=[
            pltpu.VMEM((NCHUNK, CHUNK), jnp.int32),
            pltpu.VMEM((2, NBUF, CHUNK, DH), jnp.float32),
            pltpu.SemaphoreType.DMA,
            pltpu.SemaphoreType.DMA,
        ],
        compiler_params=pltpu.CompilerParams(use_tc_tiling_on_sc=False),
    )(_gather_body)
    return f(k_tab.reshape(B * N * 2 * H, DH),
             v_tab.reshape(B * N * 2 * H, DH),
             idx.reshape(ROWS // CHUNK, CHUNK))


def _attn_body(q_ref, ks_ref, vs_ref, out_ref):
    q = q_ref[0]
    ks = ks_ref[0]
    vs = vs_ref[0]
    seg = (lax.broadcasted_iota(jnp.int32, (INNER, H), 0) // DH
           == lax.broadcasted_iota(jnp.int32, (INNER, H), 1)
           ).astype(jnp.float32)
    qk = q[:, None, :] * ks
    sim = lax.dot_general(qk.reshape(NB3 * P, INNER), seg,
                          (((1,), (0,)), ((), ())),
                          preferred_element_type=jnp.float32) * SCALE
    sim = sim.reshape(NB3, P, H)
    sim = sim - jnp.max(sim, axis=1, keepdims=True)
    e = jnp.exp(sim)
    attn = e / jnp.sum(e, axis=1, keepdims=True)
    a_exp = lax.dot_general(attn.reshape(NB3 * P, H), seg,
                            (((1,), (1,)), ((), ())),
                            preferred_element_type=jnp.float32)
    out_ref[0] = jnp.sum(a_exp.reshape(NB3, P, INNER) * vs, axis=1)


def _attn(q, ks, vs):
    grid = (B, N // NB3)
    return pl.pallas_call(
        _attn_body,
        grid=grid,
        in_specs=[
            pl.BlockSpec((1, NB3, INNER), lambda b, nb: (b, nb, 0)),
            pl.BlockSpec((1, NB3, P, INNER), lambda b, nb: (b, nb, 0, 0)),
            pl.BlockSpec((1, NB3, P, INNER), lambda b, nb: (b, nb, 0, 0)),
        ],
        out_specs=pl.BlockSpec((1, NB3, INNER), lambda b, nb: (b, nb, 0)),
        out_shape=jax.ShapeDtypeStruct((B, N, INNER), jnp.float32),
    )(q, ks.reshape(B, N, P, INNER), vs.reshape(B, N, P, INNER))


def _proj_body(a_ref, w_ref, b_ref, out_ref):
    out_ref[0] = _dot_t(a_ref[0], w_ref[...]) + b_ref[...][None, :]


def _proj(att, out_w, out_b):
    grid = (B, N // NB4)
    return pl.pallas_call(
        _proj_body,
        grid=grid,
        in_specs=[
            pl.BlockSpec((1, NB4, INNER), lambda b, nb: (b, nb, 0)),
            pl.BlockSpec((D, INNER), lambda b, nb: (0, 0)),
            pl.BlockSpec((D,), lambda b, nb: (0,)),
        ],
        out_specs=pl.BlockSpec((1, NB4, D), lambda b, nb: (b, nb, 0)),
        out_shape=jax.ShapeDtypeStruct((B, N, D), jnp.float32),
    )(att, out_w, out_b)


def kernel(x, prev_x, norm_q_g, norm_q_b, norm_k_g, norm_k_b, norm_v_g,
           norm_v_b, W_q, W_k, W_v, off_w, off_b, aw_w, aw_b, out_w, out_b):
    off_w_p = off_w.reshape(H, P, INNER).transpose(1, 0, 2).reshape(H * P, INNER)
    off_b_p = off_b.reshape(H, P).transpose(1, 0).reshape(H * P)
    q, off, idx, k_tab, v_tab = _stage1(
        x, prev_x, norm_q_g, norm_q_b, norm_k_g, norm_k_b, norm_v_g, norm_v_b,
        W_q, W_k, W_v, off_w_p, off_b_p)
    ks, vs = _gather(k_tab, v_tab, idx)
    att = _attn(q, ks, vs)
    out = _proj(att, out_w, out_b)
    offsets = off.reshape(B, N, P, H).transpose(0, 3, 2, 1)
    return (out, offsets)
